# baseline (speedup 1.0000x reference)
import functools

import jax
import jax.numpy as jnp
from jax.experimental import pallas as pl
from jax.experimental.pallas import tpu as pltpu

B_NODE = 512
TE = 256
ROW_TILE = 2048


def _ceil_to(x, m):
    return ((x + m - 1) // m) * m


def _agg_kernel(x_ref, batch_ref, vne_ref, src_ref, dst_ref, attr_ref,
                ew_ref, ebias_ref, eps_ref, i_ref, j_ref,
                zp_ref, xin_ref, *, bn, t2):
    c = pl.program_id(0)
    e = pl.program_id(1)
    N, _ = x_ref.shape
    te = attr_ref.shape[0]

    @pl.when(e == 0)
    def _init():
        G = vne_ref.shape[0]
        bm = (jax.lax.broadcasted_iota(jnp.int32, (G, N), 0)
              == batch_ref[...]).astype(jnp.bfloat16)
        xin = x_ref[...].astype(jnp.float32) + jax.lax.dot_general(
            bm, vne_ref[...], (((0,), (0,)), ((), ())),
            preferred_element_type=jnp.float32)
        xin_ref[...] = xin.astype(jnp.bfloat16)
        zp_ref[...] = jnp.where(c == 0, (1.0 + eps_ref[0, 0]) * xin, 0.0)

    t = c * t2 + e
    i = i_ref[t] * bn
    j = j_ref[t] * bn
    emb = jnp.dot(attr_ref[...], ew_ref[...],
                  preferred_element_type=jnp.float32) + ebias_ref[...]
    sub_iota = jax.lax.broadcasted_iota(jnp.int32, (bn, te), 0)
    gm = (sub_iota == src_ref[...]).astype(jnp.bfloat16)
    xblk = xin_ref[pl.ds(pl.multiple_of(i, bn), bn), :]
    xj = jax.lax.dot_general(gm, xblk, (((0,), (0,)), ((), ())),
                             preferred_element_type=jnp.float32)
    msg = jnp.maximum(xj + emb, 0.0).astype(jnp.bfloat16)
    sm = (sub_iota == dst_ref[...]).astype(jnp.bfloat16)
    jj = pl.ds(pl.multiple_of(j, bn), bn)
    zp_ref[jj, :] += jnp.dot(sm, msg, preferred_element_type=jnp.float32)


def _aggregate(x, batch_row, vne, src_row, dst_row_l, attr, ew, ebias, eps,
               i_arr, j_arr, n_tiles):
    N, D = x.shape
    G = vne.shape[0]
    K = attr.shape[1]
    t2 = n_tiles // 2
    return pl.pallas_call(
        functools.partial(_agg_kernel, bn=B_NODE, t2=t2),
        out_shape=jax.ShapeDtypeStruct((2 * N, D), jnp.float32),
        grid=(2, t2),
        in_specs=[
            pl.BlockSpec((N, D), lambda c, e: (0, 0)),
            pl.BlockSpec((1, N), lambda c, e: (0, 0)),
            pl.BlockSpec((G, D), lambda c, e: (0, 0)),
            pl.BlockSpec((1, TE), lambda c, e: (0, c * t2 + e)),
            pl.BlockSpec((1, TE), lambda c, e: (0, c * t2 + e)),
            pl.BlockSpec((TE, K), lambda c, e: (c * t2 + e, 0)),
            pl.BlockSpec((K, D), lambda c, e: (0, 0)),
            pl.BlockSpec((1, D), lambda c, e: (0, 0)),
            pl.BlockSpec(memory_space=pltpu.SMEM),
            pl.BlockSpec(memory_space=pltpu.SMEM),
            pl.BlockSpec(memory_space=pltpu.SMEM),
        ],
        out_specs=pl.BlockSpec((N, D), lambda c, e: (c, 0)),
        scratch_shapes=[pltpu.VMEM((N, D), jnp.bfloat16)],
        compiler_params=pltpu.CompilerParams(
            dimension_semantics=("parallel", "arbitrary"),
            vmem_limit_bytes=48 * 1024 * 1024),
    )(x, batch_row, vne, src_row, dst_row_l, attr, ew, ebias, eps,
      i_arr, j_arr)


def _mlp_kernel(z0_ref, z1_ref, w1_ref, t1_ref, w2_ref, t2_ref, o_ref,
                *, relu_out):
    z = (z0_ref[...] + z1_ref[...]).astype(jnp.bfloat16)
    h1 = jnp.maximum(jnp.dot(z, w1_ref[...],
                             preferred_element_type=jnp.float32) + t1_ref[...],
                     0.0)
    y = jnp.dot(h1.astype(jnp.bfloat16), w2_ref[...],
                preferred_element_type=jnp.float32) + t2_ref[...]
    if relu_out:
        y = jnp.maximum(y, 0.0)
    o_ref[...] = y.astype(o_ref.dtype)


def _mlp(zp, w1, t1, w2, t2, *, relu_out):
    twoN, D = zp.shape
    N = twoN // 2
    H = w1.shape[1]
    Dout = w2.shape[1]
    tm = ROW_TILE if N % ROW_TILE == 0 else N
    nt = N // tm
    return pl.pallas_call(
        functools.partial(_mlp_kernel, relu_out=relu_out),
        out_shape=jax.ShapeDtypeStruct((N, Dout), jnp.bfloat16),
        grid=(nt,),
        in_specs=[
            pl.BlockSpec((tm, D), lambda i: (i, 0)),
            pl.BlockSpec((tm, D), lambda i: (nt + i, 0)),
            pl.BlockSpec((D, H), lambda i: (0, 0)),
            pl.BlockSpec((1, H), lambda i: (0, 0)),
            pl.BlockSpec((H, Dout), lambda i: (0, 0)),
            pl.BlockSpec((1, Dout), lambda i: (0, 0)),
        ],
        out_specs=pl.BlockSpec((tm, Dout), lambda i: (i, 0)),
        compiler_params=pltpu.CompilerParams(
            dimension_semantics=("parallel",)),
    )(zp, zp, w1, t1, w2, t2)


def _vn_kernel(h_ref, batch_ref, counts_ref, vne_ref,
               w1_ref, t1_ref, w2_ref, t2_ref, o_ref):
    G, _ = vne_ref.shape
    N = h_ref.shape[0]
    pm = (jax.lax.broadcasted_iota(jnp.int32, (G, N), 0)
          == batch_ref[...]).astype(jnp.bfloat16)
    pooled = jnp.dot(pm, h_ref[...], preferred_element_type=jnp.float32)
    v = (pooled + (counts_ref[...] + 1.0)
         * vne_ref[...].astype(jnp.float32)).astype(jnp.bfloat16)
    v1 = jnp.maximum(jnp.dot(v, w1_ref[...],
                             preferred_element_type=jnp.float32) + t1_ref[...],
                     0.0)
    v2 = jnp.maximum(jnp.dot(v1.astype(jnp.bfloat16), w2_ref[...],
                             preferred_element_type=jnp.float32) + t2_ref[...],
                     0.0)
    o_ref[...] = v2.astype(o_ref.dtype)


def _vn_update(h, batch_row, counts, vne, w1, t1, w2, t2):
    N, D = h.shape
    G = counts.shape[0]
    H = w1.shape[1]
    return pl.pallas_call(
        _vn_kernel,
        out_shape=jax.ShapeDtypeStruct((G, D), jnp.bfloat16),
        grid=(1,),
        in_specs=[
            pl.BlockSpec((N, D), lambda n: (0, 0)),
            pl.BlockSpec((1, N), lambda n: (0, 0)),
            pl.BlockSpec((G, 1), lambda n: (0, 0)),
            pl.BlockSpec((G, D), lambda n: (0, 0)),
            pl.BlockSpec((D, H), lambda n: (0, 0)),
            pl.BlockSpec((1, H), lambda n: (0, 0)),
            pl.BlockSpec((H, D), lambda n: (0, 0)),
            pl.BlockSpec((1, D), lambda n: (0, 0)),
        ],
        out_specs=pl.BlockSpec((G, D), lambda n: (0, 0)),
        compiler_params=pltpu.CompilerParams(
            dimension_semantics=("arbitrary",)),
    )(h, batch_row, counts, vne, w1, t1, w2, t2)


def _pool_kernel(h_ref, batch_ref, counts_ref, w_ref, b_ref, o_ref):
    G = counts_ref.shape[0]
    N = h_ref.shape[0]
    pm = (jax.lax.broadcasted_iota(jnp.int32, (G, N), 0)
          == batch_ref[...]).astype(jnp.bfloat16)
    pooled = jnp.dot(pm, h_ref[...], preferred_element_type=jnp.float32)
    mean = pooled / counts_ref[...]
    o_ref[...] = (jnp.dot(mean.astype(jnp.bfloat16), w_ref[...],
                          preferred_element_type=jnp.float32)
                  + b_ref[...]).astype(o_ref.dtype)


def _pool_pred(h, batch_row, counts, w, b):
    N, D = h.shape
    G = counts.shape[0]
    C = w.shape[1]
    return pl.pallas_call(
        _pool_kernel,
        out_shape=jax.ShapeDtypeStruct((G, C), jnp.float32),
        grid=(1,),
        in_specs=[
            pl.BlockSpec((N, D), lambda n: (0, 0)),
            pl.BlockSpec((1, N), lambda n: (0, 0)),
            pl.BlockSpec((G, 1), lambda n: (0, 0)),
            pl.BlockSpec((D, C), lambda n: (0, 0)),
            pl.BlockSpec((1, C), lambda n: (0, 0)),
        ],
        out_specs=pl.BlockSpec((G, C), lambda n: (0, 0)),
        compiler_params=pltpu.CompilerParams(
            dimension_semantics=("arbitrary",)),
    )(h, batch_row, counts, w, b)


def _bucket_edges(src_col, dst_row, edge_attr_p, n_nodes):
    Ep = src_col.shape[0]
    src = src_col[:, 0]
    dst = dst_row[0, :]
    nb = n_nodes // B_NODE
    nbuck = nb * nb
    valid = (src >= 0) & (src < n_nodes) & (dst >= 0) & (dst < n_nodes)
    bucket = jnp.where(valid, (src // B_NODE) * nb + dst // B_NODE, nbuck - 1)
    eb = max(1, (Ep - 1).bit_length())
    mask = (1 << eb) - 1
    key = (bucket.astype(jnp.uint32) << eb) | jnp.arange(Ep, dtype=jnp.uint32)
    skey = key
    sb = (skey >> eb).astype(jnp.int32)
    se = (skey & mask).astype(jnp.int32)
    ustarts = jnp.searchsorted(
        sb, jnp.arange(nbuck + 1, dtype=jnp.int32)).astype(jnp.int32)
    cnts = ustarts[1:] - ustarts[:-1]
    pc = ((cnts + TE - 1) // TE) * TE
    starts = jnp.concatenate(
        [jnp.zeros((1,), jnp.int32), jnp.cumsum(pc)[:-1].astype(jnp.int32)])

    cap = _ceil_to(Ep + nbuck * (TE - 1), 2 * TE)
    n_tiles = cap // TE
    tb = jnp.clip(
        jnp.searchsorted(starts, jnp.arange(n_tiles, dtype=jnp.int32) * TE,
                         side='right') - 1, 0, nbuck - 1)
    i_arr = (tb // nb).astype(jnp.int32)
    j_arr = (tb - (tb // nb) * nb).astype(jnp.int32)

    tb_slot = jnp.repeat(tb, TE)
    within = jnp.arange(cap, dtype=jnp.int32) - starts[tb_slot]
    is_real = within < cnts[tb_slot]
    k = jnp.clip(jnp.cumsum(is_real.astype(jnp.int32)) - 1, 0, Ep - 1)
    se_slot = se[k]
    src_s = src[se_slot]
    dst_s = dst[se_slot]
    keep = is_real & valid[se_slot]
    src_l = jnp.where(keep, src_s & (B_NODE - 1), B_NODE).astype(jnp.int32)
    dst_l = jnp.where(keep, dst_s & (B_NODE - 1), B_NODE).astype(jnp.int32)
    attr_l = jnp.where(is_real[:, None], edge_attr_p[se_slot], 0)
    return (src_l.reshape(1, cap), dst_l.reshape(1, cap), attr_l,
            i_arr, j_arr, n_tiles)


def kernel(node_emb, vn_emb, pred_w, pred_b,
           l0_edge_w, l0_edge_b, l0_eps, l0_w1, l0_t1, l0_w2, l0_t2,
           l1_edge_w, l1_edge_b, l1_eps, l1_w1, l1_t1, l1_w2, l1_t2,
           l2_edge_w, l2_edge_b, l2_eps, l2_w1, l2_t1, l2_w2, l2_t2,
           l3_edge_w, l3_edge_b, l3_eps, l3_w1, l3_t1, l3_w2, l3_t2,
           l4_edge_w, l4_edge_b, l4_eps, l4_w1, l4_t1, l4_w2, l4_t2,
           v0_w1, v0_t1, v0_w2, v0_t2,
           v1_w1, v1_t1, v1_w2, v1_t2,
           v2_w1, v2_t1, v2_w2, v2_t2,
           v3_w1, v3_t1, v3_w2, v3_t2,
           src_col, dst_row, edge_attr_p, batch_col, batch_row, counts):
    N = batch_col.shape[0]
    G = counts.shape[0]
    D = node_emb.shape[1]

    layers = [
        (l0_edge_w, l0_edge_b, l0_eps, l0_w1, l0_t1, l0_w2, l0_t2),
        (l1_edge_w, l1_edge_b, l1_eps, l1_w1, l1_t1, l1_w2, l1_t2),
        (l2_edge_w, l2_edge_b, l2_eps, l2_w1, l2_t1, l2_w2, l2_t2),
        (l3_edge_w, l3_edge_b, l3_eps, l3_w1, l3_t1, l3_w2, l3_t2),
        (l4_edge_w, l4_edge_b, l4_eps, l4_w1, l4_t1, l4_w2, l4_t2),
    ]
    vn_mlps = [
        (v0_w1, v0_t1, v0_w2, v0_t2),
        (v1_w1, v1_t1, v1_w2, v1_t2),
        (v2_w1, v2_t1, v2_w2, v2_t2),
        (v3_w1, v3_t1, v3_w2, v3_t2),
    ]

    src_l, dst_l, attr_l, i_arr, j_arr, n_tiles = _bucket_edges(
        src_col, dst_row, edge_attr_p, N)

    h = jnp.broadcast_to(node_emb[0], (N, D)).astype(jnp.bfloat16)
    vne = jnp.broadcast_to(vn_emb[0], (G, D)).astype(jnp.bfloat16)

    num_layer = len(layers)
    for l, (ew, ebias, eps, w1, t1, w2, t2) in enumerate(layers):
        zp = _aggregate(h, batch_row, vne, src_l, dst_l, attr_l,
                        ew, ebias, eps, i_arr, j_arr, n_tiles)
        if l < num_layer - 1:
            vw1, vt1, vw2, vt2 = vn_mlps[l]
            vne = _vn_update(h, batch_row, counts, vne, vw1, vt1, vw2, vt2)
        h = _mlp(zp, w1, t1, w2, t2, relu_out=l < num_layer - 1)

    return _pool_pred(h, batch_row, counts, pred_w, pred_b)


# speedup vs baseline: 2.3229x; 2.3229x over previous
import functools

import jax
import jax.numpy as jnp
from jax.experimental import pallas as pl
from jax.experimental.pallas import tpu as pltpu

B_NODE = 512
TE = 256
ROW_TILE = 2048
CHUNK = 8192
K_SUB = 8


def _ceil_to(x, m):
    return ((x + m - 1) // m) * m


def _agg_kernel(x_ref, batch_ref, vne_ref, sd_ref, attr_ref,
                ew_ref, ebias_ref, eps_ref, i_ref, j_ref,
                zp_ref, xin_ref, *, bn, te, ksub, t2):
    c = pl.program_id(0)
    e = pl.program_id(1)
    N, _ = x_ref.shape

    @pl.when(e == 0)
    def _init():
        G = vne_ref.shape[0]
        bm = (jax.lax.broadcasted_iota(jnp.int32, (G, N), 0)
              == batch_ref[...]).astype(jnp.bfloat16)
        xin = x_ref[...].astype(jnp.float32) + jax.lax.dot_general(
            bm, vne_ref[...], (((0,), (0,)), ((), ())),
            preferred_element_type=jnp.float32)
        xin_ref[...] = xin.astype(jnp.bfloat16)
        zp_ref[...] = jnp.where(c == 0, (1.0 + eps_ref[0, 0]) * xin, 0.0)

    base_t = (c * t2 + e) * ksub
    sub_iota = jax.lax.broadcasted_iota(jnp.int32, (bn, te), 0)
    for k in range(ksub):
        t = base_t + k
        i = i_ref[t] * bn
        j = j_ref[t] * bn
        sd = sd_ref[:, k * te:(k + 1) * te]
        emb = jnp.dot(attr_ref[k * te:(k + 1) * te, :], ew_ref[...],
                      preferred_element_type=jnp.float32) + ebias_ref[...]
        gm = (sub_iota == (sd & 0xffff)).astype(jnp.bfloat16)
        xblk = xin_ref[pl.ds(pl.multiple_of(i, bn), bn), :]
        xj = jax.lax.dot_general(gm, xblk, (((0,), (0,)), ((), ())),
                                 preferred_element_type=jnp.float32)
        msg = jnp.maximum(xj + emb, 0.0).astype(jnp.bfloat16)
        sm = (sub_iota == (sd >> 16)).astype(jnp.bfloat16)
        jj = pl.ds(pl.multiple_of(j, bn), bn)
        zp_ref[jj, :] += jnp.dot(sm, msg, preferred_element_type=jnp.float32)


def _aggregate(x, batch_row, vne, sd_row, attr, ew, ebias, eps,
               i_arr, j_arr, n_tiles):
    N, D = x.shape
    G = vne.shape[0]
    K = attr.shape[1]
    kte = K_SUB * TE
    t2 = n_tiles // (2 * K_SUB)
    return pl.pallas_call(
        functools.partial(_agg_kernel, bn=B_NODE, te=TE, ksub=K_SUB, t2=t2),
        out_shape=jax.ShapeDtypeStruct((2 * N, D), jnp.float32),
        grid=(2, t2),
        in_specs=[
            pl.BlockSpec((N, D), lambda c, e: (0, 0)),
            pl.BlockSpec((1, N), lambda c, e: (0, 0)),
            pl.BlockSpec((G, D), lambda c, e: (0, 0)),
            pl.BlockSpec((1, kte), lambda c, e: (0, c * t2 + e)),
            pl.BlockSpec((kte, K), lambda c, e: (c * t2 + e, 0)),
            pl.BlockSpec((K, D), lambda c, e: (0, 0)),
            pl.BlockSpec((1, D), lambda c, e: (0, 0)),
            pl.BlockSpec(memory_space=pltpu.SMEM),
            pl.BlockSpec(memory_space=pltpu.SMEM),
            pl.BlockSpec(memory_space=pltpu.SMEM),
        ],
        out_specs=pl.BlockSpec((N, D), lambda c, e: (c, 0)),
        scratch_shapes=[pltpu.VMEM((N, D), jnp.bfloat16)],
        compiler_params=pltpu.CompilerParams(
            dimension_semantics=("parallel", "arbitrary"),
            vmem_limit_bytes=48 * 1024 * 1024),
    )(x, batch_row, vne, sd_row, attr, ew, ebias, eps,
      i_arr, j_arr)


def _mlp_kernel(z0_ref, z1_ref, w1_ref, t1_ref, w2_ref, t2_ref, o_ref,
                *, relu_out):
    z = (z0_ref[...] + z1_ref[...]).astype(jnp.bfloat16)
    h1 = jnp.maximum(jnp.dot(z, w1_ref[...],
                             preferred_element_type=jnp.float32) + t1_ref[...],
                     0.0)
    y = jnp.dot(h1.astype(jnp.bfloat16), w2_ref[...],
                preferred_element_type=jnp.float32) + t2_ref[...]
    if relu_out:
        y = jnp.maximum(y, 0.0)
    o_ref[...] = y.astype(o_ref.dtype)


def _mlp(zp, w1, t1, w2, t2, *, relu_out):
    twoN, D = zp.shape
    N = twoN // 2
    H = w1.shape[1]
    Dout = w2.shape[1]
    tm = ROW_TILE if N % ROW_TILE == 0 else N
    nt = N // tm
    return pl.pallas_call(
        functools.partial(_mlp_kernel, relu_out=relu_out),
        out_shape=jax.ShapeDtypeStruct((N, Dout), jnp.bfloat16),
        grid=(nt,),
        in_specs=[
            pl.BlockSpec((tm, D), lambda i: (i, 0)),
            pl.BlockSpec((tm, D), lambda i: (nt + i, 0)),
            pl.BlockSpec((D, H), lambda i: (0, 0)),
            pl.BlockSpec((1, H), lambda i: (0, 0)),
            pl.BlockSpec((H, Dout), lambda i: (0, 0)),
            pl.BlockSpec((1, Dout), lambda i: (0, 0)),
        ],
        out_specs=pl.BlockSpec((tm, Dout), lambda i: (i, 0)),
        compiler_params=pltpu.CompilerParams(
            dimension_semantics=("parallel",)),
    )(zp, zp, w1, t1, w2, t2)


def _vn_kernel(h_ref, batch_ref, counts_ref, vne_ref,
               w1_ref, t1_ref, w2_ref, t2_ref, o_ref):
    G, _ = vne_ref.shape
    N = h_ref.shape[0]
    pm = (jax.lax.broadcasted_iota(jnp.int32, (G, N), 0)
          == batch_ref[...]).astype(jnp.bfloat16)
    pooled = jnp.dot(pm, h_ref[...], preferred_element_type=jnp.float32)
    v = (pooled + (counts_ref[...] + 1.0)
         * vne_ref[...].astype(jnp.float32)).astype(jnp.bfloat16)
    v1 = jnp.maximum(jnp.dot(v, w1_ref[...],
                             preferred_element_type=jnp.float32) + t1_ref[...],
                     0.0)
    v2 = jnp.maximum(jnp.dot(v1.astype(jnp.bfloat16), w2_ref[...],
                             preferred_element_type=jnp.float32) + t2_ref[...],
                     0.0)
    o_ref[...] = v2.astype(o_ref.dtype)


def _vn_update(h, batch_row, counts, vne, w1, t1, w2, t2):
    N, D = h.shape
    G = counts.shape[0]
    H = w1.shape[1]
    return pl.pallas_call(
        _vn_kernel,
        out_shape=jax.ShapeDtypeStruct((G, D), jnp.bfloat16),
        grid=(1,),
        in_specs=[
            pl.BlockSpec((N, D), lambda n: (0, 0)),
            pl.BlockSpec((1, N), lambda n: (0, 0)),
            pl.BlockSpec((G, 1), lambda n: (0, 0)),
            pl.BlockSpec((G, D), lambda n: (0, 0)),
            pl.BlockSpec((D, H), lambda n: (0, 0)),
            pl.BlockSpec((1, H), lambda n: (0, 0)),
            pl.BlockSpec((H, D), lambda n: (0, 0)),
            pl.BlockSpec((1, D), lambda n: (0, 0)),
        ],
        out_specs=pl.BlockSpec((G, D), lambda n: (0, 0)),
        compiler_params=pltpu.CompilerParams(
            dimension_semantics=("arbitrary",)),
    )(h, batch_row, counts, vne, w1, t1, w2, t2)


def _pool_kernel(h_ref, batch_ref, counts_ref, w_ref, b_ref, o_ref):
    G = counts_ref.shape[0]
    N = h_ref.shape[0]
    pm = (jax.lax.broadcasted_iota(jnp.int32, (G, N), 0)
          == batch_ref[...]).astype(jnp.bfloat16)
    pooled = jnp.dot(pm, h_ref[...], preferred_element_type=jnp.float32)
    mean = pooled / counts_ref[...]
    o_ref[...] = (jnp.dot(mean.astype(jnp.bfloat16), w_ref[...],
                          preferred_element_type=jnp.float32)
                  + b_ref[...]).astype(o_ref.dtype)


def _pool_pred(h, batch_row, counts, w, b):
    N, D = h.shape
    G = counts.shape[0]
    C = w.shape[1]
    return pl.pallas_call(
        _pool_kernel,
        out_shape=jax.ShapeDtypeStruct((G, C), jnp.float32),
        grid=(1,),
        in_specs=[
            pl.BlockSpec((N, D), lambda n: (0, 0)),
            pl.BlockSpec((1, N), lambda n: (0, 0)),
            pl.BlockSpec((G, 1), lambda n: (0, 0)),
            pl.BlockSpec((D, C), lambda n: (0, 0)),
            pl.BlockSpec((1, C), lambda n: (0, 0)),
        ],
        out_specs=pl.BlockSpec((G, C), lambda n: (0, 0)),
        compiler_params=pltpu.CompilerParams(
            dimension_semantics=("arbitrary",)),
    )(h, batch_row, counts, w, b)


def _plan_kernel(bkt_ref, rank_ref, hist_ref, bsm, rsm, cnt, sem,
                 *, ch, nbuck, s2):
    c = pl.program_id(0)
    s = pl.program_id(1)
    chunk = c * s2 + s

    @pl.when(s == 0)
    def _zero():
        def zero(i, _):
            cnt[i] = 0
            return 0
        jax.lax.fori_loop(0, nbuck, zero, 0)

    cp_in = pltpu.make_async_copy(bkt_ref.at[chunk], bsm, sem)
    cp_in.start()
    cp_in.wait()

    def body(e, _):
        b = bsm[e]
        r = cnt[b]
        rsm[e] = r
        cnt[b] = r + 1
        return 0
    jax.lax.fori_loop(0, ch, body, 0)

    cp_out = pltpu.make_async_copy(rsm, rank_ref.at[chunk], sem)
    cp_out.start()
    cp_out.wait()

    @pl.when(s == s2 - 1)
    def _flush():
        cp_h = pltpu.make_async_copy(cnt, hist_ref.at[c], sem)
        cp_h.start()
        cp_h.wait()


def _plan(bkt2d, nbuck):
    nch, ch = bkt2d.shape
    s2 = nch // 2
    return pl.pallas_call(
        functools.partial(_plan_kernel, ch=ch, nbuck=nbuck, s2=s2),
        out_shape=(jax.ShapeDtypeStruct((nch, ch), jnp.int32),
                   jax.ShapeDtypeStruct((2, nbuck), jnp.int32)),
        grid=(2, s2),
        in_specs=[pl.BlockSpec(memory_space=pl.ANY)],
        out_specs=(pl.BlockSpec(memory_space=pl.ANY),
                   pl.BlockSpec(memory_space=pl.ANY)),
        scratch_shapes=[pltpu.SMEM((ch,), jnp.int32),
                        pltpu.SMEM((ch,), jnp.int32),
                        pltpu.SMEM((nbuck,), jnp.int32),
                        pltpu.SemaphoreType.DMA],
        compiler_params=pltpu.CompilerParams(
            dimension_semantics=("parallel", "arbitrary")),
    )(bkt2d)


def _bucket_edges(src_col, dst_row, edge_attr_p, n_nodes):
    Ep = src_col.shape[0]
    src = src_col[:, 0]
    dst = dst_row[0, :]
    nb = n_nodes // B_NODE
    nbuck = nb * nb
    valid = (src >= 0) & (src < n_nodes) & (dst >= 0) & (dst < n_nodes)
    bucket = jnp.where(valid, (src // B_NODE) * nb + dst // B_NODE,
                       nbuck - 1).astype(jnp.int32)

    ch = CHUNK
    ep2 = _ceil_to(Ep, 2 * ch)
    bkt2d = jnp.full((ep2,), nbuck - 1,
                     jnp.int32).at[:Ep].set(bucket).reshape(ep2 // ch, ch)
    rank2d, hist = _plan(bkt2d, nbuck)
    rank = rank2d.reshape(ep2)[:Ep]
    cnts = hist[0] + hist[1]
    pc = ((cnts + TE - 1) // TE) * TE
    starts = jnp.concatenate(
        [jnp.zeros((1,), jnp.int32), jnp.cumsum(pc)[:-1].astype(jnp.int32)])
    tab = jnp.concatenate([starts, starts + hist[0]])
    idx = bucket + jnp.where(jnp.arange(Ep) < ep2 // 2, 0, nbuck)
    pos = tab[idx] + rank

    cap = _ceil_to(ep2 + nbuck * (TE - 1), 2 * K_SUB * TE)
    n_tiles = cap // TE
    tb = jnp.clip(
        jnp.searchsorted(starts, jnp.arange(n_tiles, dtype=jnp.int32) * TE,
                         side='right') - 1, 0, nbuck - 1)
    i_arr = (tb // nb).astype(jnp.int32)
    j_arr = (tb - (tb // nb) * nb).astype(jnp.int32)

    sentinel = B_NODE | (B_NODE << 16)
    packed = jnp.where(valid,
                       (src & (B_NODE - 1)) | ((dst & (B_NODE - 1)) << 16),
                       sentinel).astype(jnp.int32)
    sd_l = jnp.full((cap,), sentinel, jnp.int32).at[pos].set(packed)
    attr_l = jnp.zeros((cap, edge_attr_p.shape[1]),
                       edge_attr_p.dtype).at[pos].set(edge_attr_p)
    return sd_l.reshape(1, cap), attr_l, i_arr, j_arr, n_tiles


def kernel(node_emb, vn_emb, pred_w, pred_b,
           l0_edge_w, l0_edge_b, l0_eps, l0_w1, l0_t1, l0_w2, l0_t2,
           l1_edge_w, l1_edge_b, l1_eps, l1_w1, l1_t1, l1_w2, l1_t2,
           l2_edge_w, l2_edge_b, l2_eps, l2_w1, l2_t1, l2_w2, l2_t2,
           l3_edge_w, l3_edge_b, l3_eps, l3_w1, l3_t1, l3_w2, l3_t2,
           l4_edge_w, l4_edge_b, l4_eps, l4_w1, l4_t1, l4_w2, l4_t2,
           v0_w1, v0_t1, v0_w2, v0_t2,
           v1_w1, v1_t1, v1_w2, v1_t2,
           v2_w1, v2_t1, v2_w2, v2_t2,
           v3_w1, v3_t1, v3_w2, v3_t2,
           src_col, dst_row, edge_attr_p, batch_col, batch_row, counts):
    N = batch_col.shape[0]
    G = counts.shape[0]
    D = node_emb.shape[1]

    layers = [
        (l0_edge_w, l0_edge_b, l0_eps, l0_w1, l0_t1, l0_w2, l0_t2),
        (l1_edge_w, l1_edge_b, l1_eps, l1_w1, l1_t1, l1_w2, l1_t2),
        (l2_edge_w, l2_edge_b, l2_eps, l2_w1, l2_t1, l2_w2, l2_t2),
        (l3_edge_w, l3_edge_b, l3_eps, l3_w1, l3_t1, l3_w2, l3_t2),
        (l4_edge_w, l4_edge_b, l4_eps, l4_w1, l4_t1, l4_w2, l4_t2),
    ]
    vn_mlps = [
        (v0_w1, v0_t1, v0_w2, v0_t2),
        (v1_w1, v1_t1, v1_w2, v1_t2),
        (v2_w1, v2_t1, v2_w2, v2_t2),
        (v3_w1, v3_t1, v3_w2, v3_t2),
    ]

    sd_l, attr_l, i_arr, j_arr, n_tiles = _bucket_edges(
        src_col, dst_row, edge_attr_p, N)

    h = jnp.broadcast_to(node_emb[0], (N, D)).astype(jnp.bfloat16)
    vne = jnp.broadcast_to(vn_emb[0], (G, D)).astype(jnp.bfloat16)

    num_layer = len(layers)
    for l, (ew, ebias, eps, w1, t1, w2, t2) in enumerate(layers):
        zp = _aggregate(h, batch_row, vne, sd_l, attr_l,
                        ew, ebias, eps, i_arr, j_arr, n_tiles)
        if l < num_layer - 1:
            vw1, vt1, vw2, vt2 = vn_mlps[l]
            vne = _vn_update(h, batch_row, counts, vne, vw1, vt1, vw2, vt2)
        h = _mlp(zp, w1, t1, w2, t2, relu_out=l < num_layer - 1)

    return _pool_pred(h, batch_row, counts, pred_w, pred_b)


# speedup vs baseline: 2.3320x; 1.0039x over previous
import functools

import jax
import jax.numpy as jnp
from jax.experimental import pallas as pl
from jax.experimental.pallas import tpu as pltpu

B_NODE = 512
TE = 256
ROW_TILE = 2048
CHUNK = 8192
K_SUB = 8


def _ceil_to(x, m):
    return ((x + m - 1) // m) * m


def _agg_kernel(x_ref, batch_ref, vne_ref, sd_ref, attr_ref,
                ew_ref, ebias_ref, eps_ref, i_ref, j_ref,
                zp_ref, xin_ref, *, bn, te, ksub, t2):
    c = pl.program_id(0)
    e = pl.program_id(1)
    N, _ = x_ref.shape

    @pl.when(e == 0)
    def _init():
        G = vne_ref.shape[0]
        bm = (jax.lax.broadcasted_iota(jnp.int32, (G, N), 0)
              == batch_ref[...]).astype(jnp.bfloat16)
        xin = x_ref[...].astype(jnp.float32) + jax.lax.dot_general(
            bm, vne_ref[...], (((0,), (0,)), ((), ())),
            preferred_element_type=jnp.float32)
        xin_ref[...] = xin.astype(jnp.bfloat16)
        zp_ref[...] = jnp.where(c == 0, (1.0 + eps_ref[0, 0]) * xin, 0.0)

    base_t = (c * t2 + e) * ksub
    sub_iota = jax.lax.broadcasted_iota(jnp.int32, (bn, te), 0)
    for k in range(ksub):
        t = base_t + k
        i = i_ref[t] * bn
        j = j_ref[t] * bn
        sd = sd_ref[:, k * te:(k + 1) * te]
        a = attr_ref[k * te:(k + 1) * te, :]
        evens = jax.lax.bitcast_convert_type(a << 16, jnp.float32)
        odds = jax.lax.bitcast_convert_type(a & jnp.int32(-65536), jnp.float32)
        attr8 = jnp.concatenate([evens, odds], axis=1).astype(jnp.bfloat16)
        emb = jnp.dot(attr8, ew_ref[...],
                      preferred_element_type=jnp.float32) + ebias_ref[...]
        gm = (sub_iota == (sd & 0xffff)).astype(jnp.bfloat16)
        xblk = xin_ref[pl.ds(pl.multiple_of(i, bn), bn), :]
        xj = jax.lax.dot_general(gm, xblk, (((0,), (0,)), ((), ())),
                                 preferred_element_type=jnp.float32)
        msg = jnp.maximum(xj + emb, 0.0).astype(jnp.bfloat16)
        sm = (sub_iota == (sd >> 16)).astype(jnp.bfloat16)
        jj = pl.ds(pl.multiple_of(j, bn), bn)
        zp_ref[jj, :] += jnp.dot(sm, msg, preferred_element_type=jnp.float32)


def _aggregate(x, batch_row, vne, sd_row, attr, ew, ebias, eps,
               i_arr, j_arr, n_tiles):
    N, D = x.shape
    G = vne.shape[0]
    K = attr.shape[1]
    Kw = ew.shape[0]
    kte = K_SUB * TE
    t2 = n_tiles // (2 * K_SUB)
    return pl.pallas_call(
        functools.partial(_agg_kernel, bn=B_NODE, te=TE, ksub=K_SUB, t2=t2),
        out_shape=jax.ShapeDtypeStruct((2 * N, D), jnp.float32),
        grid=(2, t2),
        in_specs=[
            pl.BlockSpec((N, D), lambda c, e: (0, 0)),
            pl.BlockSpec((1, N), lambda c, e: (0, 0)),
            pl.BlockSpec((G, D), lambda c, e: (0, 0)),
            pl.BlockSpec((1, kte), lambda c, e: (0, c * t2 + e)),
            pl.BlockSpec((kte, K), lambda c, e: (c * t2 + e, 0)),
            pl.BlockSpec((Kw, D), lambda c, e: (0, 0)),
            pl.BlockSpec((1, D), lambda c, e: (0, 0)),
            pl.BlockSpec(memory_space=pltpu.SMEM),
            pl.BlockSpec(memory_space=pltpu.SMEM),
            pl.BlockSpec(memory_space=pltpu.SMEM),
        ],
        out_specs=pl.BlockSpec((N, D), lambda c, e: (c, 0)),
        scratch_shapes=[pltpu.VMEM((N, D), jnp.bfloat16)],
        compiler_params=pltpu.CompilerParams(
            dimension_semantics=("parallel", "arbitrary"),
            vmem_limit_bytes=48 * 1024 * 1024),
    )(x, batch_row, vne, sd_row, attr, ew, ebias, eps,
      i_arr, j_arr)


def _mlp_kernel(z0_ref, z1_ref, w1_ref, t1_ref, w2_ref, t2_ref, o_ref,
                *, relu_out):
    z = (z0_ref[...] + z1_ref[...]).astype(jnp.bfloat16)
    h1 = jnp.maximum(jnp.dot(z, w1_ref[...],
                             preferred_element_type=jnp.float32) + t1_ref[...],
                     0.0)
    y = jnp.dot(h1.astype(jnp.bfloat16), w2_ref[...],
                preferred_element_type=jnp.float32) + t2_ref[...]
    if relu_out:
        y = jnp.maximum(y, 0.0)
    o_ref[...] = y.astype(o_ref.dtype)


def _mlp(zp, w1, t1, w2, t2, *, relu_out):
    twoN, D = zp.shape
    N = twoN // 2
    H = w1.shape[1]
    Dout = w2.shape[1]
    tm = ROW_TILE if N % ROW_TILE == 0 else N
    nt = N // tm
    return pl.pallas_call(
        functools.partial(_mlp_kernel, relu_out=relu_out),
        out_shape=jax.ShapeDtypeStruct((N, Dout), jnp.bfloat16),
        grid=(nt,),
        in_specs=[
            pl.BlockSpec((tm, D), lambda i: (i, 0)),
            pl.BlockSpec((tm, D), lambda i: (nt + i, 0)),
            pl.BlockSpec((D, H), lambda i: (0, 0)),
            pl.BlockSpec((1, H), lambda i: (0, 0)),
            pl.BlockSpec((H, Dout), lambda i: (0, 0)),
            pl.BlockSpec((1, Dout), lambda i: (0, 0)),
        ],
        out_specs=pl.BlockSpec((tm, Dout), lambda i: (i, 0)),
        compiler_params=pltpu.CompilerParams(
            dimension_semantics=("parallel",)),
    )(zp, zp, w1, t1, w2, t2)


def _vn_kernel(h_ref, batch_ref, counts_ref, vne_ref,
               w1_ref, t1_ref, w2_ref, t2_ref, o_ref):
    G, _ = vne_ref.shape
    N = h_ref.shape[0]
    pm = (jax.lax.broadcasted_iota(jnp.int32, (G, N), 0)
          == batch_ref[...]).astype(jnp.bfloat16)
    pooled = jnp.dot(pm, h_ref[...], preferred_element_type=jnp.float32)
    v = (pooled + (counts_ref[...] + 1.0)
         * vne_ref[...].astype(jnp.float32)).astype(jnp.bfloat16)
    v1 = jnp.maximum(jnp.dot(v, w1_ref[...],
                             preferred_element_type=jnp.float32) + t1_ref[...],
                     0.0)
    v2 = jnp.maximum(jnp.dot(v1.astype(jnp.bfloat16), w2_ref[...],
                             preferred_element_type=jnp.float32) + t2_ref[...],
                     0.0)
    o_ref[...] = v2.astype(o_ref.dtype)


def _vn_update(h, batch_row, counts, vne, w1, t1, w2, t2):
    N, D = h.shape
    G = counts.shape[0]
    H = w1.shape[1]
    return pl.pallas_call(
        _vn_kernel,
        out_shape=jax.ShapeDtypeStruct((G, D), jnp.bfloat16),
        grid=(1,),
        in_specs=[
            pl.BlockSpec((N, D), lambda n: (0, 0)),
            pl.BlockSpec((1, N), lambda n: (0, 0)),
            pl.BlockSpec((G, 1), lambda n: (0, 0)),
            pl.BlockSpec((G, D), lambda n: (0, 0)),
            pl.BlockSpec((D, H), lambda n: (0, 0)),
            pl.BlockSpec((1, H), lambda n: (0, 0)),
            pl.BlockSpec((H, D), lambda n: (0, 0)),
            pl.BlockSpec((1, D), lambda n: (0, 0)),
        ],
        out_specs=pl.BlockSpec((G, D), lambda n: (0, 0)),
        compiler_params=pltpu.CompilerParams(
            dimension_semantics=("arbitrary",)),
    )(h, batch_row, counts, vne, w1, t1, w2, t2)


def _pool_kernel(h_ref, batch_ref, counts_ref, w_ref, b_ref, o_ref):
    G = counts_ref.shape[0]
    N = h_ref.shape[0]
    pm = (jax.lax.broadcasted_iota(jnp.int32, (G, N), 0)
          == batch_ref[...]).astype(jnp.bfloat16)
    pooled = jnp.dot(pm, h_ref[...], preferred_element_type=jnp.float32)
    mean = pooled / counts_ref[...]
    o_ref[...] = (jnp.dot(mean.astype(jnp.bfloat16), w_ref[...],
                          preferred_element_type=jnp.float32)
                  + b_ref[...]).astype(o_ref.dtype)


def _pool_pred(h, batch_row, counts, w, b):
    N, D = h.shape
    G = counts.shape[0]
    C = w.shape[1]
    return pl.pallas_call(
        _pool_kernel,
        out_shape=jax.ShapeDtypeStruct((G, C), jnp.float32),
        grid=(1,),
        in_specs=[
            pl.BlockSpec((N, D), lambda n: (0, 0)),
            pl.BlockSpec((1, N), lambda n: (0, 0)),
            pl.BlockSpec((G, 1), lambda n: (0, 0)),
            pl.BlockSpec((D, C), lambda n: (0, 0)),
            pl.BlockSpec((1, C), lambda n: (0, 0)),
        ],
        out_specs=pl.BlockSpec((G, C), lambda n: (0, 0)),
        compiler_params=pltpu.CompilerParams(
            dimension_semantics=("arbitrary",)),
    )(h, batch_row, counts, w, b)


def _plan_kernel(bkt_ref, rank_ref, hist_ref, bsm, rsm, cnt, sem,
                 *, ch, nbuck, s2):
    c = pl.program_id(0)
    s = pl.program_id(1)
    chunk = c * s2 + s

    @pl.when(s == 0)
    def _zero():
        def zero(i, _):
            cnt[i] = 0
            return 0
        jax.lax.fori_loop(0, nbuck, zero, 0)

    cp_in = pltpu.make_async_copy(bkt_ref.at[chunk], bsm, sem)
    cp_in.start()
    cp_in.wait()

    def body(e, _):
        b = bsm[e]
        r = cnt[b]
        rsm[e] = r
        cnt[b] = r + 1
        return 0
    jax.lax.fori_loop(0, ch, body, 0)

    cp_out = pltpu.make_async_copy(rsm, rank_ref.at[chunk], sem)
    cp_out.start()
    cp_out.wait()

    @pl.when(s == s2 - 1)
    def _flush():
        cp_h = pltpu.make_async_copy(cnt, hist_ref.at[c], sem)
        cp_h.start()
        cp_h.wait()


def _plan(bkt2d, nbuck):
    nch, ch = bkt2d.shape
    s2 = nch // 2
    return pl.pallas_call(
        functools.partial(_plan_kernel, ch=ch, nbuck=nbuck, s2=s2),
        out_shape=(jax.ShapeDtypeStruct((nch, ch), jnp.int32),
                   jax.ShapeDtypeStruct((2, nbuck), jnp.int32)),
        grid=(2, s2),
        in_specs=[pl.BlockSpec(memory_space=pl.ANY)],
        out_specs=(pl.BlockSpec(memory_space=pl.ANY),
                   pl.BlockSpec(memory_space=pl.ANY)),
        scratch_shapes=[pltpu.SMEM((ch,), jnp.int32),
                        pltpu.SMEM((ch,), jnp.int32),
                        pltpu.SMEM((nbuck,), jnp.int32),
                        pltpu.SemaphoreType.DMA],
        compiler_params=pltpu.CompilerParams(
            dimension_semantics=("parallel", "arbitrary")),
    )(bkt2d)


def _bucket_edges(src_col, dst_row, edge_attr_p, n_nodes):
    Ep = src_col.shape[0]
    src = src_col[:, 0]
    dst = dst_row[0, :]
    nb = n_nodes // B_NODE
    nbuck = nb * nb
    valid = (src >= 0) & (src < n_nodes) & (dst >= 0) & (dst < n_nodes)
    bucket = jnp.where(valid, (src // B_NODE) * nb + dst // B_NODE,
                       nbuck - 1).astype(jnp.int32)

    ch = CHUNK
    ep2 = _ceil_to(Ep, 2 * ch)
    bkt2d = jnp.full((ep2,), nbuck - 1,
                     jnp.int32).at[:Ep].set(bucket).reshape(ep2 // ch, ch)
    rank2d, hist = _plan(bkt2d, nbuck)
    rank = rank2d.reshape(ep2)[:Ep]
    cnts = hist[0] + hist[1]
    pc = ((cnts + TE - 1) // TE) * TE
    starts = jnp.concatenate(
        [jnp.zeros((1,), jnp.int32), jnp.cumsum(pc)[:-1].astype(jnp.int32)])
    tab = jnp.concatenate([starts, starts + hist[0]])
    idx = bucket + jnp.where(jnp.arange(Ep) < ep2 // 2, 0, nbuck)
    pos = tab[idx] + rank

    cap = _ceil_to(ep2 + nbuck * (TE - 1), 2 * K_SUB * TE)
    n_tiles = cap // TE
    tb = jnp.clip(
        jnp.searchsorted(starts, jnp.arange(n_tiles, dtype=jnp.int32) * TE,
                         side='right') - 1, 0, nbuck - 1)
    i_arr = (tb // nb).astype(jnp.int32)
    j_arr = (tb - (tb // nb) * nb).astype(jnp.int32)

    sentinel = B_NODE | (B_NODE << 16)
    packed = jnp.where(valid,
                       (src & (B_NODE - 1)) | ((dst & (B_NODE - 1)) << 16),
                       sentinel).astype(jnp.int32)
    sd_l = jnp.full((cap,), sentinel, jnp.int32).at[pos].set(packed)
    K8 = edge_attr_p.shape[1]
    attr_i32 = jax.lax.bitcast_convert_type(
        edge_attr_p.reshape(Ep, K8 // 2, 2), jnp.int32)
    attr_l = jnp.zeros((cap, K8 // 2), jnp.int32).at[pos].set(attr_i32)
    return sd_l.reshape(1, cap), attr_l, i_arr, j_arr, n_tiles


def kernel(node_emb, vn_emb, pred_w, pred_b,
           l0_edge_w, l0_edge_b, l0_eps, l0_w1, l0_t1, l0_w2, l0_t2,
           l1_edge_w, l1_edge_b, l1_eps, l1_w1, l1_t1, l1_w2, l1_t2,
           l2_edge_w, l2_edge_b, l2_eps, l2_w1, l2_t1, l2_w2, l2_t2,
           l3_edge_w, l3_edge_b, l3_eps, l3_w1, l3_t1, l3_w2, l3_t2,
           l4_edge_w, l4_edge_b, l4_eps, l4_w1, l4_t1, l4_w2, l4_t2,
           v0_w1, v0_t1, v0_w2, v0_t2,
           v1_w1, v1_t1, v1_w2, v1_t2,
           v2_w1, v2_t1, v2_w2, v2_t2,
           v3_w1, v3_t1, v3_w2, v3_t2,
           src_col, dst_row, edge_attr_p, batch_col, batch_row, counts):
    N = batch_col.shape[0]
    G = counts.shape[0]
    D = node_emb.shape[1]

    layers = [
        (l0_edge_w, l0_edge_b, l0_eps, l0_w1, l0_t1, l0_w2, l0_t2),
        (l1_edge_w, l1_edge_b, l1_eps, l1_w1, l1_t1, l1_w2, l1_t2),
        (l2_edge_w, l2_edge_b, l2_eps, l2_w1, l2_t1, l2_w2, l2_t2),
        (l3_edge_w, l3_edge_b, l3_eps, l3_w1, l3_t1, l3_w2, l3_t2),
        (l4_edge_w, l4_edge_b, l4_eps, l4_w1, l4_t1, l4_w2, l4_t2),
    ]
    vn_mlps = [
        (v0_w1, v0_t1, v0_w2, v0_t2),
        (v1_w1, v1_t1, v1_w2, v1_t2),
        (v2_w1, v2_t1, v2_w2, v2_t2),
        (v3_w1, v3_t1, v3_w2, v3_t2),
    ]

    sd_l, attr_l, i_arr, j_arr, n_tiles = _bucket_edges(
        src_col, dst_row, edge_attr_p, N)

    h = jnp.broadcast_to(node_emb[0], (N, D)).astype(jnp.bfloat16)
    vne = jnp.broadcast_to(vn_emb[0], (G, D)).astype(jnp.bfloat16)

    num_layer = len(layers)
    kdim = l0_edge_w.shape[0]
    ew_perm = jnp.asarray(list(range(0, kdim, 2)) + list(range(1, kdim, 2)))
    for l, (ew, ebias, eps, w1, t1, w2, t2) in enumerate(layers):
        zp = _aggregate(h, batch_row, vne, sd_l, attr_l,
                        ew[ew_perm], ebias, eps, i_arr, j_arr, n_tiles)
        if l < num_layer - 1:
            vw1, vt1, vw2, vt2 = vn_mlps[l]
            vne = _vn_update(h, batch_row, counts, vne, vw1, vt1, vw2, vt2)
        h = _mlp(zp, w1, t1, w2, t2, relu_out=l < num_layer - 1)

    return _pool_pred(h, batch_row, counts, pred_w, pred_b)
